# baseline (speedup 1.0000x reference)
import jax
import jax.numpy as jnp
from jax import lax
from jax.experimental import pallas as pl
from jax.experimental.pallas import tpu as pltpu
from jax.experimental.pallas import tpu_sc as plsc

N_NODES = 10000
N_PAD = 10240
D = 128
DH = D // 2
N_EDGES = 320000
CHUNK = 128
NT = 16
CH_PER_T = 160
NCH_TOT = NT * (CH_PER_T + 4)
DUMMY = N_NODES + 200
RPW = N_PAD // NT
BLK = 1024


def _make_sc_agg(with_counts: bool):
    mesh = plsc.VectorSubcoreMesh(core_axis_name="c", subcore_axis_name="s")
    out_type = [jax.ShapeDtypeStruct((2 * N_PAD, DH), jnp.float32)]
    if with_counts:
        out_type.append(jax.ShapeDtypeStruct((NT * N_PAD,), jnp.float32))
    scratch = [
        pltpu.VMEM_SHARED((N_PAD, DH), jnp.float32),
        pltpu.VMEM_SHARED((N_PAD, DH), jnp.float32),
        [pltpu.VMEM((CHUNK, DH), jnp.float32) for _ in range(4)],
        [pltpu.VMEM((2, CHUNK), jnp.int32) for _ in range(6)],
        [pltpu.SemaphoreType.DMA for _ in range(4)],
        [pltpu.SemaphoreType.DMA for _ in range(4)],
        [pltpu.SemaphoreType.DMA for _ in range(6)],
    ]
    if with_counts:
        scratch.append(pltpu.VMEM((N_PAD,), jnp.float32))

    def body(z, edges, *rest):
        if with_counts:
            out, cnt_out = rest[0], rest[1]
            acc, tab, rows, ei, gsem, ssem, isem, cnt_v = rest[2:]
        else:
            out = rest[0]
            cnt_out = cnt_v = None
            acc, tab, rows, ei, gsem, ssem, isem = rest[1:]

        cid = lax.axis_index("c")
        sid = lax.axis_index("s")

        zeros16 = jnp.zeros((16,), jnp.float32)

        pltpu.sync_copy(z.at[pl.ds(cid * N_PAD + sid * RPW, RPW)],
                        tab.at[pl.ds(sid * RPW, RPW)])

        @pl.loop(0, CHUNK)
        def _zero_zbuf(i):
            for j in range(DH // 16):
                rows[0][i, pl.ds(j * 16, 16)] = zeros16

        for r in range(RPW // CHUNK):
            pltpu.sync_copy(rows[0],
                            acc.at[pl.ds(sid * RPW + r * CHUNK, CHUNK)])

        if with_counts:
            @pl.loop(0, N_PAD // 16)
            def _zero_cnt(i):
                cnt_v[pl.ds(i * 16, 16)] = zeros16

        plsc.subcore_barrier()

        ones16 = jnp.ones((16,), jnp.float32)

        def idx_start(cn, p=None):
            p = cn if p is None else p
            pltpu.async_copy(edges.at[sid + NT * cn], ei[p % 6], isem[p % 6])

        def idx_wait(c):
            pltpu.make_async_copy(edges.at[0], ei[c % 6], isem[c % 6]).wait()

        def gather_start(c):
            pltpu.async_copy(tab.at[ei[c % 6].at[0]], rows[c % 4],
                             gsem[c % 4])

        def gather_wait(c):
            pltpu.make_async_copy(tab.at[ei[c % 6].at[0]], rows[c % 4],
                                  gsem[c % 4]).wait()

        def scatter_start(c):
            pltpu.async_copy(rows[c % 4], acc.at[ei[c % 6].at[1]],
                             ssem[c % 4], add=True)
            if with_counts:
                @pl.when(cid == 0)
                def _():
                    for j in range(CHUNK // 16):
                        plsc.addupdate_scatter(
                            cnt_v, [ei[c % 6][1, pl.ds(j * 16, 16)]], ones16)

        def scatter_wait(c):
            pltpu.make_async_copy(rows[c % 4], acc.at[ei[c % 6].at[1]],
                                  ssem[c % 4]).wait()

        def steady(c, cc=None):
            idx_wait(c + 2)
            scatter_wait(c - 2)
            gather_start(c + 2)
            idx_start((c if cc is None else cc) + 4, c + 4)
            gather_wait(c)
            scatter_start(c)

        for c in range(4):
            idx_start(c)
        idx_wait(0)
        gather_start(0)
        idx_wait(1)
        gather_start(1)
        for c in range(2):
            idx_wait(c + 2)
            gather_start(c + 2)
            idx_start(c + 4)
            gather_wait(c)
            scatter_start(c)

        @pl.loop(0, (CH_PER_T - 4) // 12)
        def _chunks(h):
            for j in range(12):
                c = 2 + j
                steady(c, cc=12 * h + c)

        for c in range(CH_PER_T - 2, CH_PER_T):
            scatter_wait(c - 2)
            gather_wait(c)
            scatter_start(c)
        scatter_wait(CH_PER_T - 2)
        scatter_wait(CH_PER_T - 1)

        plsc.subcore_barrier()
        pltpu.sync_copy(acc.at[pl.ds(sid * RPW, RPW)],
                        out.at[pl.ds(cid * N_PAD + sid * RPW, RPW)])
        if with_counts:
            @pl.when(cid == 0)
            def _():
                pltpu.sync_copy(cnt_v, cnt_out.at[pl.ds(sid * N_PAD, N_PAD)])

    return pl.kernel(body, out_type=tuple(out_type), mesh=mesh,
                     scratch_types=tuple(scratch),
                     compiler_params=pltpu.CompilerParams(
                         needs_layout_passes=False,
                         use_tc_tiling_on_sc=False))


_sc_agg_counts = _make_sc_agg(True)
_sc_agg = _make_sc_agg(False)


def _tc_linear2(x, Wa, Wb):
    def body(x_ref, wa_ref, wb_ref, z_ref, y_ref):
        xb = x_ref[...]
        dn = (((1,), (1,)), ((), ()))
        z = lax.dot_general(xb, wa_ref[...], dn,
                            preferred_element_type=jnp.float32)
        z_ref[0] = z[:, :DH]
        z_ref[1] = z[:, DH:]
        y_ref[...] = lax.dot_general(xb, wb_ref[...], dn,
                                     preferred_element_type=jnp.float32)

    return pl.pallas_call(
        body,
        grid=(N_PAD // BLK,),
        in_specs=[pl.BlockSpec((BLK, D), lambda i: (i, 0)),
                  pl.BlockSpec((D, D), lambda i: (0, 0)),
                  pl.BlockSpec((D, D), lambda i: (0, 0))],
        out_specs=[pl.BlockSpec((2, BLK, DH), lambda i: (0, i, 0)),
                   pl.BlockSpec((BLK, D), lambda i: (i, 0))],
        out_shape=[jax.ShapeDtypeStruct((2, N_PAD, DH), jnp.float32),
                   jax.ShapeDtypeStruct((N_PAD, D), jnp.float32)],
    )(x, Wa, Wb)


def _tc_mid(psum, cnt_p, y1, b1l, W2l, W2r):
    def body(p_ref, c_ref, y_ref, b_ref, wa_ref, wb_ref, z_ref, y2_ref):
        cnt = jnp.sum(c_ref[...], axis=0)
        s = jnp.concatenate([p_ref[0], p_ref[1]], axis=1)
        mean = s / jnp.clip(cnt, 1.0, None)[:, None]
        h = jnp.maximum(mean + b_ref[...] + y_ref[...], 0.0)
        dn = (((1,), (1,)), ((), ()))
        z = lax.dot_general(h, wa_ref[...], dn,
                            preferred_element_type=jnp.float32)
        z_ref[0] = z[:, :DH]
        z_ref[1] = z[:, DH:]
        y2_ref[...] = lax.dot_general(h, wb_ref[...], dn,
                                      preferred_element_type=jnp.float32)

    return pl.pallas_call(
        body,
        grid=(N_PAD // BLK,),
        in_specs=[pl.BlockSpec((2, BLK, DH), lambda i: (0, i, 0)),
                  pl.BlockSpec((NT, BLK), lambda i: (0, i)),
                  pl.BlockSpec((BLK, D), lambda i: (i, 0)),
                  pl.BlockSpec((1, D), lambda i: (0, 0)),
                  pl.BlockSpec((D, D), lambda i: (0, 0)),
                  pl.BlockSpec((D, D), lambda i: (0, 0))],
        out_specs=[pl.BlockSpec((2, BLK, DH), lambda i: (0, i, 0)),
                   pl.BlockSpec((BLK, D), lambda i: (i, 0))],
        out_shape=[jax.ShapeDtypeStruct((2, N_PAD, DH), jnp.float32),
                   jax.ShapeDtypeStruct((N_PAD, D), jnp.float32)],
    )(psum, cnt_p, y1, b1l, W2l, W2r)


def _tc_out(psum, cnt_p, y2, b2l):
    def body(p_ref, c_ref, y_ref, b_ref, o_ref):
        cnt = jnp.sum(c_ref[...], axis=0)
        s = jnp.concatenate([p_ref[0], p_ref[1]], axis=1)
        mean = s / jnp.clip(cnt, 1.0, None)[:, None]
        o_ref[...] = mean + b_ref[...] + y_ref[...]

    return pl.pallas_call(
        body,
        grid=(N_PAD // BLK,),
        in_specs=[pl.BlockSpec((2, BLK, DH), lambda i: (0, i, 0)),
                  pl.BlockSpec((NT, BLK), lambda i: (0, i)),
                  pl.BlockSpec((BLK, D), lambda i: (i, 0)),
                  pl.BlockSpec((1, D), lambda i: (0, 0))],
        out_specs=pl.BlockSpec((BLK, D), lambda i: (i, 0)),
        out_shape=jax.ShapeDtypeStruct((N_PAD, D), jnp.float32),
    )(psum, cnt_p, y2, b2l)


def kernel(x, edge_index, W1l, b1l, W1r, W2l, b2l, W2r):
    n = x.shape[0]
    e = edge_index.shape[1]
    e_alloc = NCH_TOT * CHUNK
    src = edge_index[0].astype(jnp.int32)
    dst = edge_index[1].astype(jnp.int32)
    src_p = jnp.concatenate([src, jnp.zeros((e_alloc - e,), jnp.int32)])
    dst_p = jnp.concatenate([dst, jnp.full((e_alloc - e,), DUMMY, jnp.int32)])
    edges = jnp.stack([src_p.reshape(NCH_TOT, CHUNK),
                       dst_p.reshape(NCH_TOT, CHUNK)], axis=1)
    x_p = jnp.pad(x.astype(jnp.float32), ((0, N_PAD - n), (0, 0)))

    z1, y1 = _tc_linear2(x_p, W1l, W1r)
    p1_flat, cnt_flat = _sc_agg_counts(z1.reshape(2 * N_PAD, DH), edges)
    p1 = p1_flat.reshape(2, N_PAD, DH)
    cnt_p = cnt_flat.reshape(NT, N_PAD)
    z2, y2 = _tc_mid(p1, cnt_p, y1, b1l.reshape(1, D), W2l, W2r)
    p2 = _sc_agg(z2.reshape(2 * N_PAD, DH), edges)[0].reshape(2, N_PAD, DH)
    out = _tc_out(p2, cnt_p, y2, b2l.reshape(1, D))
    return out[:n]
